# baseline (speedup 1.0000x reference)
import functools

import jax
import jax.numpy as jnp
import numpy as np
from jax import lax
from jax.experimental import pallas as pl
from jax.experimental.pallas import tpu as pltpu
from jax.experimental.pallas import tpu_sc as plsc

N = 10000
E = 320000
D = 128
XP = 16
TW = 160
GW = TW // 2

NC, NS, L = 2, 16, 16
NW = NC * NS
NSLICE = 5
ES = E // NSLICE
EPW = ES // NW
CH = 80
NCHUNK = EPW // CH
RPS = N // NS

GES = ES * GW // 128
GCH = CH * GW // 128
CVR = ES // 8
CCH = CH // 8
TWW = TW // 2

_f32 = jnp.float32
_bf16 = jnp.bfloat16
_i32 = jnp.int32
_mesh = plsc.VectorSubcoreMesh(core_axis_name="c", subcore_axis_name="s")
_sc_params = pltpu.CompilerParams(use_tc_tiling_on_sc=False)
_sc_gather_params = pltpu.CompilerParams(use_tc_tiling_on_sc=False,
                                         needs_layout_passes=False)

_PERM = np.concatenate([np.arange(0, D, 2), np.arange(1, D, 2)])


def _tables_body(h_ref, xlo_ref, xhi_ref, wlor_ref, whir_ref, wloc_ref,
                 whic_ref, tr_ref, tc_ref):
    h = h_ref[...]
    nb = h.shape[0]
    pad = jnp.zeros((nb, TWW - D // 2 - XP // 2), _f32)

    def bf16_bits(w_ref, x_ref):
        t = jnp.concatenate(
            [jnp.dot(h, w_ref[...], preferred_element_type=_f32),
             x_ref[...], pad], axis=1)
        b = jax.lax.bitcast_convert_type(t, _i32)
        r = b + jnp.int32(0x7FFF) + ((b >> 16) & 1)
        return jax.lax.shift_right_logical(r, 16)

    tr_ref[...] = (bf16_bits(whir_ref, xhi_ref) << 16) | \
        (bf16_bits(wlor_ref, xlo_ref) & jnp.int32(0xFFFF))
    tc_ref[...] = (bf16_bits(whic_ref, xhi_ref) << 16) | \
        (bf16_bits(wloc_ref, xlo_ref) & jnp.int32(0xFFFF))


def _make_tables(h, xlo, xhi, wlor, whir, wloc, whic):
    nb = 1000
    grid = N // nb
    return pl.pallas_call(
        _tables_body,
        grid=(grid,),
        in_specs=[
            pl.BlockSpec((nb, D), lambda i: (i, 0)),
            pl.BlockSpec((nb, XP // 2), lambda i: (i, 0)),
            pl.BlockSpec((nb, XP // 2), lambda i: (i, 0)),
            pl.BlockSpec((D, D // 2), lambda i: (0, 0)),
            pl.BlockSpec((D, D // 2), lambda i: (0, 0)),
            pl.BlockSpec((D, D // 2), lambda i: (0, 0)),
            pl.BlockSpec((D, D // 2), lambda i: (0, 0)),
        ],
        out_specs=[
            pl.BlockSpec((nb, TWW), lambda i: (i, 0)),
            pl.BlockSpec((nb, TWW), lambda i: (i, 0)),
        ],
        out_shape=[
            jax.ShapeDtypeStruct((N, TWW), _i32),
            jax.ShapeDtypeStruct((N, TWW), _i32),
        ],
    )(h, xlo, xhi, wlor, whir, wloc, whic)


def _make_sc_gather(s):
    @functools.partial(
        pl.kernel,
        out_type=jax.ShapeDtypeStruct((ES, GW), _i32),
        mesh=_mesh,
        scratch_types=[
            pltpu.VMEM((2, CH), _i32),
            pltpu.VMEM((2, CH), _i32),
            pltpu.VMEM((2, CH, TWW), _i32),
            pltpu.VMEM((2, CH, TWW), _i32),
            pltpu.VMEM((2, CH, GW), _i32),
            pltpu.SemaphoreType.DMA,
            pltpu.SemaphoreType.DMA,
            pltpu.SemaphoreType.DMA,
            pltpu.SemaphoreType.DMA,
            pltpu.SemaphoreType.DMA,
            pltpu.SemaphoreType.DMA,
        ],
        compiler_params=_sc_gather_params,
    )
    def _sc_gather(tr_hbm, tc_hbm, ei_hbm, g_hbm,
                   idxr, idxc, abuf, bbuf, gbuf,
                   sa0, sa1, sb0, sb1, w0, w1):
        wid = lax.axis_index("s") * NC + lax.axis_index("c")
        sa = (sa0, sa1)
        sb = (sb0, sb1)
        ws = (w0, w1)

        def ebase(ci):
            return wid * EPW + ci * CH

        def issue(ci, b):
            base = ebase(ci)
            pltpu.sync_copy(ei_hbm.at[0, pl.ds(s * ES + base, CH)],
                            idxr.at[b])
            pltpu.sync_copy(ei_hbm.at[1, pl.ds(s * ES + base, CH)],
                            idxc.at[b])
            pltpu.async_copy(tr_hbm.at[idxr.at[b]], abuf.at[b], sa[b])
            pltpu.async_copy(tc_hbm.at[idxc.at[b]], bbuf.at[b], sb[b])

        def wait_gather(b):
            pltpu.make_async_copy(tr_hbm.at[idxr.at[b]], abuf.at[b],
                                  sa[b]).wait()
            pltpu.make_async_copy(tc_hbm.at[idxc.at[b]], bbuf.at[b],
                                  sb[b]).wait()

        def wait_write(ci, b):
            pltpu.make_async_copy(gbuf.at[b], g_hbm.at[pl.ds(ebase(ci), CH)],
                                  ws[b]).wait()

        def compute(b):
            @pl.loop(0, CH)
            def _row(i):
                for j in range(TW // 32):
                    sl = pl.ds(16 * j, 16)
                    av = plsc.bitcast(abuf[b, i, sl], _bf16)
                    bv = plsc.bitcast(bbuf[b, i, sl], _bf16)
                    v = av + bv if j < D // 32 else av - bv
                    gbuf[b, i, sl] = plsc.bitcast(v, _i32)

        issue(0, 0)
        issue(1, 1)

        @pl.loop(0, NCHUNK - 1, step=2)
        def _chunk(ci):
            for b in (0, 1):
                cur = ci + b
                wait_gather(b)

                @pl.when(cur >= 2)
                def _():
                    wait_write(cur - 2, b)

                compute(b)

                @pl.when(cur + 2 < NCHUNK)
                def _():
                    issue(cur + 2, b)

                pltpu.async_copy(gbuf.at[b],
                                 g_hbm.at[pl.ds(ebase(cur), CH)], ws[b])

        last = NCHUNK - 1
        wait_gather(0)
        wait_write(last - 2, 0)
        compute(0)
        pltpu.sync_copy(gbuf.at[0], g_hbm.at[pl.ds(ebase(last), CH)])
        wait_write(last - 1, 1)

    return _sc_gather


_sc_gathers = [_make_sc_gather(s) for s in range(NSLICE)]


def _edge_body(gi_ref, ea_ref, wea_ref, wrad_ref, be1_ref, we2_ref,
               be2_ref, wa_ref, ba_ref, wc1_ref, bc1_ref, wc2_ref,
               m_ref, cv_ref):
    gi = gi_ref[...]
    eb = gi.shape[0]
    lo = jax.lax.bitcast_convert_type(gi << 16, _f32)
    hi = jax.lax.bitcast_convert_type(gi & jnp.int32(-65536), _f32)
    g = jnp.concatenate([lo[:, :D // 2], hi[:, :D // 2]], axis=1)
    d = jnp.concatenate([lo[:, D // 2:D // 2 + XP // 2],
                         hi[:, D // 2:D // 2 + XP // 2]], axis=1)
    ea_term = jnp.dot(ea_ref[...], wea_ref[...], preferred_element_type=_f32)
    radial = jnp.sum(d * d, axis=1, keepdims=True)
    pre = g + ea_term + radial * wrad_ref[...] + be1_ref[...]
    m1 = jax.nn.silu(pre)
    m2 = jax.nn.silu(jnp.dot(m1, we2_ref[...], preferred_element_type=_f32)
                     + be2_ref[...])
    att = jax.nn.sigmoid(jnp.dot(m2, wa_ref[...], preferred_element_type=_f32)
                         + ba_ref[...])
    m = m2 * att
    m_ref[...] = m
    cu = jnp.dot(jax.nn.silu(jnp.dot(m, wc1_ref[...],
                                     preferred_element_type=_f32)
                             + bc1_ref[...]),
                 wc2_ref[...], preferred_element_type=_f32)
    cv = d * cu
    lane = lax.broadcasted_iota(jnp.int32, cv.shape, 1)
    cv_ref[...] = jnp.where(lane == 3, 1.0, cv)


def _edge_mlp(s, gi, ea_t, wea, wrad, be1, we2, be2, wa, ba, wc1, bc1, wc2):
    eb = 2000
    grid = ES // eb
    off = s * (ES // eb)
    full = lambda shp: pl.BlockSpec(shp, lambda i: tuple(0 for _ in shp))
    return pl.pallas_call(
        _edge_body,
        grid=(grid,),
        in_specs=[
            pl.BlockSpec((eb, GW), lambda i: (i, 0)),
            pl.BlockSpec((eb, 4), lambda i: (i + off, 0)),
            full((4, D)), full((1, D)), full((1, D)), full((D, D)),
            full((1, D)), full((D, 1)), full((1, 1)), full((D, D)),
            full((1, D)), full((D, 1)),
        ],
        out_specs=[
            pl.BlockSpec((eb, D), lambda i: (i, 0)),
            pl.BlockSpec((eb, XP), lambda i: (i, 0)),
        ],
        out_shape=[
            jax.ShapeDtypeStruct((ES, D), _f32),
            jax.ShapeDtypeStruct((ES, XP), _f32),
        ],
    )(gi, ea_t, wea, wrad, be1, we2, be2, wa, ba, wc1, bc1, wc2)


def _make_sc_scatter(slice_ids):
    nsl = len(slice_ids)

    def body(*refs):
        m_s = refs[0:nsl]
        cv_s = refs[nsl:2 * nsl]
        ei_hbm, inith, initc, aggh_hbm, aggc_hbm = refs[2 * nsl:2 * nsl + 5]
        mbuf, cvbuf, idx, acch, accc, l0, l1 = refs[2 * nsl + 5:]
        cid = lax.axis_index("c")
        sid = lax.axis_index("s")
        wid = sid * NC + cid
        rows = pl.ds(sid * RPS, RPS)
        ls = (l0, l1)

        def issue(p, ci, b):
            base = wid * EPW + ci * CH
            gbase = slice_ids[p] * ES + base
            pltpu.async_copy(ei_hbm.at[0, pl.ds(gbase, CH)], idx.at[b],
                             ls[b])
            pltpu.async_copy(m_s[p].at[pl.ds(base, CH)], mbuf.at[b], ls[b])
            pltpu.async_copy(cv_s[p].at[pl.ds(base, CH)], cvbuf.at[b],
                             ls[b])

        def wait_loads(p, ci, b):
            base = wid * EPW + ci * CH
            gbase = slice_ids[p] * ES + base
            pltpu.make_async_copy(ei_hbm.at[0, pl.ds(gbase, CH)], idx.at[b],
                                  ls[b]).wait()
            pltpu.make_async_copy(m_s[p].at[pl.ds(base, CH)], mbuf.at[b],
                                  ls[b]).wait()
            pltpu.make_async_copy(cv_s[p].at[pl.ds(base, CH)],
                                  cvbuf.at[b], ls[b]).wait()

        def scat(b):
            pltpu.sync_copy(mbuf.at[b], acch.at[idx.at[b]], add=True)
            pltpu.sync_copy(cvbuf.at[b], accc.at[idx.at[b]], add=True)

        issue(0, 0, 0)
        issue(0, 1, 1)
        pltpu.sync_copy(inith.at[cid, rows], acch.at[rows])
        pltpu.sync_copy(initc.at[cid, rows], accc.at[rows])
        plsc.subcore_barrier()

        for p in range(nsl):
            @pl.loop(0, NCHUNK - 1, step=2)
            def _chunk(ci, p=p):
                for b in (0, 1):
                    bb = (b + p) % 2
                    cur = ci + b
                    wait_loads(p, cur, bb)
                    scat(bb)
                    nxt = cur + 2
                    if p + 1 < nsl:
                        @pl.when(nxt < NCHUNK)
                        def _():
                            issue(p, nxt, bb)

                        @pl.when(nxt >= NCHUNK)
                        def _():
                            issue(p + 1, nxt - NCHUNK, bb)
                    else:
                        @pl.when(nxt < NCHUNK)
                        def _():
                            issue(p, nxt, bb)

            last = NCHUNK - 1
            bb = p % 2
            wait_loads(p, last, bb)
            scat(bb)
            if p + 1 < nsl:
                issue(p + 1, 1, bb)

        plsc.subcore_barrier()
        pltpu.sync_copy(acch.at[rows], aggh_hbm.at[cid, rows])
        pltpu.sync_copy(accc.at[rows], aggc_hbm.at[cid, rows])

    return functools.partial(
        pl.kernel,
        out_type=(jax.ShapeDtypeStruct((NC, N, D), _f32),
                  jax.ShapeDtypeStruct((NC, N, XP), _f32)),
        mesh=_mesh,
        scratch_types=[
            pltpu.VMEM((2, CH, D), _f32),
            pltpu.VMEM((2, CH, XP), _f32),
            pltpu.VMEM((2, CH), _i32),
            pltpu.VMEM_SHARED((N, D), _f32),
            pltpu.VMEM_SHARED((N, XP), _f32),
            pltpu.SemaphoreType.DMA,
            pltpu.SemaphoreType.DMA,
        ],
        compiler_params=_sc_params,
    )(body)


_sc_scatter_a = _make_sc_scatter((0, 1, 2))
_sc_scatter_b = _make_sc_scatter((3, 4))


def _node_body(h_ref, aggp_ref, cp_ref, xp_ref, wn1h_ref, wn1a_ref, bn1_ref,
               wn2_ref, bn2_ref, ho_ref, xo_ref):
    h = h_ref[...]
    agg = aggp_ref[0] + aggp_ref[1]
    u = jax.nn.silu(jnp.dot(h, wn1h_ref[...], preferred_element_type=_f32)
                    + jnp.dot(agg, wn1a_ref[...], preferred_element_type=_f32)
                    + bn1_ref[...])
    ho_ref[...] = h + jnp.dot(u, wn2_ref[...], preferred_element_type=_f32) \
        + bn2_ref[...]
    s = cp_ref[0] + cp_ref[1]
    cnt = jnp.maximum(s[:, 3:4], 1.0)
    coord = jnp.concatenate(
        [s[:, 0:1], s[:, 8:9], s[:, 1:2],
         jnp.zeros((s.shape[0], XP - 3), _f32)], axis=1)
    xo_ref[...] = xp_ref[...] + coord / cnt


def _node_update(h, aggp, cp, xp, wn1h, wn1a, bn1, wn2, bn2):
    nb = 1000
    grid = N // nb
    full = lambda shp: pl.BlockSpec(shp, lambda i: tuple(0 for _ in shp))
    return pl.pallas_call(
        _node_body,
        grid=(grid,),
        in_specs=[
            pl.BlockSpec((nb, D), lambda i: (i, 0)),
            pl.BlockSpec((NC, nb, D), lambda i: (0, i, 0)),
            pl.BlockSpec((NC, nb, XP), lambda i: (0, i, 0)),
            pl.BlockSpec((nb, XP), lambda i: (i, 0)),
            full((D, D)), full((D, D)), full((1, D)), full((D, D)),
            full((1, D)),
        ],
        out_specs=[
            pl.BlockSpec((nb, D), lambda i: (i, 0)),
            pl.BlockSpec((nb, XP), lambda i: (i, 0)),
        ],
        out_shape=[
            jax.ShapeDtypeStruct((N, D), _f32),
            jax.ShapeDtypeStruct((N, XP), _f32),
        ],
    )(h, aggp, cp, xp, wn1h, wn1a, bn1, wn2, bn2)


def kernel(h, x, edge_index, edge_attr, W_e1, b_e1, W_e2, b_e2, W_n1, b_n1,
           W_n2, b_n2, W_c1, b_c1, W_c2, W_a, b_a):
    xp = jnp.pad(x, ((0, 0), (0, XP - 3)))
    xlo = jnp.zeros((N, XP // 2), _f32).at[:, 0].set(x[:, 0]) \
        .at[:, 1].set(x[:, 2])
    xhi = jnp.zeros((N, XP // 2), _f32).at[:, 0].set(x[:, 1])

    whr = W_e1[:D]
    whc = W_e1[D:2 * D]
    wrad = W_e1[2 * D:2 * D + 1][:, _PERM]
    wea = W_e1[2 * D + 1:][:, _PERM]
    be1 = b_e1[_PERM].reshape(1, D)
    we2 = W_e2[_PERM, :]

    tr, tc = _make_tables(h, xlo, xhi, whr[:, 0::2], whr[:, 1::2],
                          whc[:, 0::2], whc[:, 1::2])
    ms, cvs = [], []
    for s in range(NSLICE):
        gi = _sc_gathers[s](tr, tc, edge_index)
        m, cv = _edge_mlp(s, gi, edge_attr, wea, wrad, be1, we2,
                          b_e2.reshape(1, D), W_a, b_a.reshape(1, 1),
                          W_c1, b_c1.reshape(1, D), W_c2)
        ms.append(m)
        cvs.append(cv)
    zh = jnp.zeros((NC, N, D), _f32)
    zc = jnp.zeros((NC, N, XP), _f32)
    pa_h, pa_c = _sc_scatter_a(ms[0], ms[1], ms[2], cvs[0], cvs[1], cvs[2],
                               edge_index, zh, zc)
    aggp, cp = _sc_scatter_b(ms[3], ms[4], cvs[3], cvs[4],
                             edge_index, pa_h, pa_c)
    ho, xo = _node_update(h, aggp, cp, xp, W_n1[:D], W_n1[D:],
                          b_n1.reshape(1, D), W_n2, b_n2.reshape(1, D))
    return ho, xo[:, :3]


# speedup vs baseline: 1.0158x; 1.0158x over previous
import functools

import jax
import jax.numpy as jnp
import numpy as np
from jax import lax
from jax.experimental import pallas as pl
from jax.experimental.pallas import tpu as pltpu
from jax.experimental.pallas import tpu_sc as plsc

N = 10000
E = 320000
D = 128
XP = 16
TW = 160
GW = TW // 2

NC, NS, L = 2, 16, 16
NW = NC * NS
NSLICE = 5
ES = E // NSLICE
EPW = ES // NW
CH = 80
NCHUNK = EPW // CH
RPS = N // NS

_f32 = jnp.float32
_bf16 = jnp.bfloat16
_i32 = jnp.int32
_mesh = plsc.VectorSubcoreMesh(core_axis_name="c", subcore_axis_name="s")
_sc_params = pltpu.CompilerParams(use_tc_tiling_on_sc=False)
_sc_gather_params = pltpu.CompilerParams(use_tc_tiling_on_sc=False,
                                         needs_layout_passes=False)

_PERM = np.concatenate([np.arange(0, D, 2), np.arange(1, D, 2)])


def _tables_body(h_ref, xp_ref, whr_ref, whc_ref, tr_ref, tc_ref):
    h = h_ref[...]
    xp = xp_ref[...]
    pad = jnp.zeros((h.shape[0], TW - D - XP), _f32)
    tr_ref[...] = jnp.concatenate(
        [jnp.dot(h, whr_ref[...], preferred_element_type=_f32), xp, pad],
        axis=1).astype(_bf16)
    tc_ref[...] = jnp.concatenate(
        [jnp.dot(h, whc_ref[...], preferred_element_type=_f32), xp, pad],
        axis=1).astype(_bf16)


def _make_tables(h, xp, whr, whc):
    nb = 1000
    grid = N // nb
    return pl.pallas_call(
        _tables_body,
        grid=(grid,),
        in_specs=[
            pl.BlockSpec((nb, D), lambda i: (i, 0)),
            pl.BlockSpec((nb, XP), lambda i: (i, 0)),
            pl.BlockSpec((D, D), lambda i: (0, 0)),
            pl.BlockSpec((D, D), lambda i: (0, 0)),
        ],
        out_specs=[
            pl.BlockSpec((nb, TW), lambda i: (i, 0)),
            pl.BlockSpec((nb, TW), lambda i: (i, 0)),
        ],
        out_shape=[
            jax.ShapeDtypeStruct((N, TW), _bf16),
            jax.ShapeDtypeStruct((N, TW), _bf16),
        ],
    )(h, xp, whr, whc)


def _make_sc_gather(s):
    @functools.partial(
        pl.kernel,
        out_type=jax.ShapeDtypeStruct((ES, GW), _i32),
        mesh=_mesh,
        scratch_types=[
            pltpu.VMEM((2, CH), _i32),
            pltpu.VMEM((2, CH), _i32),
            pltpu.VMEM((2, CH, TW), _bf16),
            pltpu.VMEM((2, CH, TW), _bf16),
            pltpu.VMEM((2, CH, GW), _i32),
            pltpu.SemaphoreType.DMA,
            pltpu.SemaphoreType.DMA,
            pltpu.SemaphoreType.DMA,
            pltpu.SemaphoreType.DMA,
            pltpu.SemaphoreType.DMA,
            pltpu.SemaphoreType.DMA,
        ],
        compiler_params=_sc_gather_params,
    )
    def _sc_gather(tr_hbm, tc_hbm, ei_hbm, g_hbm,
                   idxr, idxc, abuf, bbuf, gbuf,
                   sa0, sa1, sb0, sb1, w0, w1):
        wid = lax.axis_index("s") * NC + lax.axis_index("c")
        sa = (sa0, sa1)
        sb = (sb0, sb1)
        ws = (w0, w1)

        def ebase(ci):
            return wid * EPW + ci * CH

        def issue(ci, b):
            base = ebase(ci)
            pltpu.sync_copy(ei_hbm.at[0, pl.ds(s * ES + base, CH)],
                            idxr.at[b])
            pltpu.sync_copy(ei_hbm.at[1, pl.ds(s * ES + base, CH)],
                            idxc.at[b])
            pltpu.async_copy(tr_hbm.at[idxr.at[b]], abuf.at[b], sa[b])
            pltpu.async_copy(tc_hbm.at[idxc.at[b]], bbuf.at[b], sb[b])

        def wait_gather(b):
            pltpu.make_async_copy(tr_hbm.at[idxr.at[b]], abuf.at[b],
                                  sa[b]).wait()
            pltpu.make_async_copy(tc_hbm.at[idxc.at[b]], bbuf.at[b],
                                  sb[b]).wait()

        def wait_write(ci, b):
            base = ebase(ci)
            pltpu.make_async_copy(gbuf.at[b], g_hbm.at[pl.ds(base, CH)],
                                  ws[b]).wait()

        def compute(b):
            @pl.loop(0, CH)
            def _row(i):
                for j in range(TW // 32):
                    sl = pl.ds(32 * j, 32)
                    if j < D // 32:
                        v = abuf[b, i, sl] + bbuf[b, i, sl]
                    else:
                        v = abuf[b, i, sl] - bbuf[b, i, sl]
                    gbuf[b, i, pl.ds(16 * j, 16)] = plsc.bitcast(v, _i32)

        issue(0, 0)
        issue(1, 1)

        @pl.loop(0, NCHUNK - 1, step=2)
        def _chunk(ci):
            for b in (0, 1):
                cur = ci + b
                wait_gather(b)

                @pl.when(cur >= 2)
                def _():
                    wait_write(cur - 2, b)

                compute(b)

                @pl.when(cur + 2 < NCHUNK)
                def _():
                    issue(cur + 2, b)

                pltpu.async_copy(gbuf.at[b],
                                 g_hbm.at[pl.ds(ebase(cur), CH)], ws[b])

        last = NCHUNK - 1
        wait_gather(0)
        wait_write(last - 2, 0)
        compute(0)
        pltpu.sync_copy(gbuf.at[0], g_hbm.at[pl.ds(ebase(last), CH)])
        wait_write(last - 1, 1)

    return _sc_gather


_sc_gathers = [_make_sc_gather(s) for s in range(NSLICE)]


def _edge_body(gi_ref, ea_ref, wea_ref, wrad_ref, be1_ref, we2_ref,
               be2_ref, wa_ref, ba_ref, wc1_ref, bc1_ref, wc2_ref,
               m_ref, cv_ref):
    gi = gi_ref[...]
    lo = jax.lax.bitcast_convert_type(gi << 16, _f32)
    hi = jax.lax.bitcast_convert_type(gi & jnp.int32(-65536), _f32)
    g = jnp.concatenate([lo[:, :D // 2], hi[:, :D // 2]], axis=1)
    d = jnp.concatenate([lo[:, D // 2:D // 2 + XP // 2],
                         hi[:, D // 2:D // 2 + XP // 2]], axis=1)
    ea = ea_ref[...]
    radial = jnp.sum(d * d, axis=1, keepdims=True)
    pre = (g + jnp.dot(ea, wea_ref[...], preferred_element_type=_f32)
           + radial * wrad_ref[...] + be1_ref[...])
    m1 = jax.nn.silu(pre)
    m2 = jax.nn.silu(jnp.dot(m1, we2_ref[...], preferred_element_type=_f32)
                     + be2_ref[...])
    att = jax.nn.sigmoid(jnp.dot(m2, wa_ref[...], preferred_element_type=_f32)
                         + ba_ref[...])
    m = m2 * att
    m_ref[...] = m
    cu = jnp.dot(jax.nn.silu(jnp.dot(m, wc1_ref[...],
                                     preferred_element_type=_f32)
                             + bc1_ref[...]),
                 wc2_ref[...], preferred_element_type=_f32)
    cv = d * cu
    lane = lax.broadcasted_iota(jnp.int32, cv.shape, 1)
    cv_ref[...] = jnp.where(lane == 3, 1.0, cv)


def _edge_mlp(s, gi, ea, wea, wrad, be1, we2, be2, wa, ba, wc1, bc1, wc2):
    eb = 2000
    grid = ES // eb
    off = s * (ES // eb)
    full = lambda shp: pl.BlockSpec(shp, lambda i: tuple(0 for _ in shp))
    return pl.pallas_call(
        _edge_body,
        grid=(grid,),
        in_specs=[
            pl.BlockSpec((eb, GW), lambda i: (i, 0)),
            pl.BlockSpec((eb, 4), lambda i: (i + off, 0)),
            full((4, D)), full((1, D)), full((1, D)), full((D, D)),
            full((1, D)), full((D, 1)), full((1, 1)), full((D, D)),
            full((1, D)), full((D, 1)),
        ],
        out_specs=[
            pl.BlockSpec((eb, D), lambda i: (i, 0)),
            pl.BlockSpec((eb, XP), lambda i: (i, 0)),
        ],
        out_shape=[
            jax.ShapeDtypeStruct((ES, D), _f32),
            jax.ShapeDtypeStruct((ES, XP), _f32),
        ],
    )(gi, ea, wea, wrad, be1, we2, be2, wa, ba, wc1, bc1, wc2)


def _make_sc_scatter(slice_ids):
    nsl = len(slice_ids)

    def body(*refs):
        m_s = refs[0:nsl]
        cv_s = refs[nsl:2 * nsl]
        ei_hbm, inith, initc, aggh_hbm, aggc_hbm = refs[2 * nsl:2 * nsl + 5]
        mbuf, cvbuf, idx, acch, accc, l0, l1 = refs[2 * nsl + 5:]
        cid = lax.axis_index("c")
        sid = lax.axis_index("s")
        wid = sid * NC + cid
        rows = pl.ds(sid * RPS, RPS)
        ls = (l0, l1)

        def issue(p, ci, b):
            base = wid * EPW + ci * CH
            gbase = slice_ids[p] * ES + base
            pltpu.async_copy(ei_hbm.at[0, pl.ds(gbase, CH)], idx.at[b],
                             ls[b])
            pltpu.async_copy(m_s[p].at[pl.ds(base, CH)], mbuf.at[b], ls[b])
            pltpu.async_copy(cv_s[p].at[pl.ds(base, CH)], cvbuf.at[b],
                             ls[b])

        def wait_loads(p, ci, b):
            base = wid * EPW + ci * CH
            gbase = slice_ids[p] * ES + base
            pltpu.make_async_copy(ei_hbm.at[0, pl.ds(gbase, CH)], idx.at[b],
                                  ls[b]).wait()
            pltpu.make_async_copy(m_s[p].at[pl.ds(base, CH)], mbuf.at[b],
                                  ls[b]).wait()
            pltpu.make_async_copy(cv_s[p].at[pl.ds(base, CH)], cvbuf.at[b],
                                  ls[b]).wait()

        def scat(b):
            pltpu.sync_copy(mbuf.at[b], acch.at[idx.at[b]], add=True)
            pltpu.sync_copy(cvbuf.at[b], accc.at[idx.at[b]], add=True)

        issue(0, 0, 0)
        issue(0, 1, 1)
        pltpu.sync_copy(inith.at[cid, rows], acch.at[rows])
        pltpu.sync_copy(initc.at[cid, rows], accc.at[rows])
        plsc.subcore_barrier()

        for p in range(nsl):
            @pl.loop(0, NCHUNK - 1, step=2)
            def _chunk(ci, p=p):
                for b in (0, 1):
                    bb = (b + p) % 2
                    cur = ci + b
                    wait_loads(p, cur, bb)
                    scat(bb)
                    nxt = cur + 2
                    if p + 1 < nsl:
                        @pl.when(nxt < NCHUNK)
                        def _():
                            issue(p, nxt, bb)

                        @pl.when(nxt >= NCHUNK)
                        def _():
                            issue(p + 1, nxt - NCHUNK, bb)
                    else:
                        @pl.when(nxt < NCHUNK)
                        def _():
                            issue(p, nxt, bb)

            last = NCHUNK - 1
            bb = p % 2
            wait_loads(p, last, bb)
            scat(bb)
            if p + 1 < nsl:
                issue(p + 1, 1, bb)

        plsc.subcore_barrier()
        pltpu.sync_copy(acch.at[rows], aggh_hbm.at[cid, rows])
        pltpu.sync_copy(accc.at[rows], aggc_hbm.at[cid, rows])

    return functools.partial(
        pl.kernel,
        out_type=(jax.ShapeDtypeStruct((NC, N, D), _f32),
                  jax.ShapeDtypeStruct((NC, N, XP), _f32)),
        mesh=_mesh,
        scratch_types=[
            pltpu.VMEM((2, CH, D), _f32),
            pltpu.VMEM((2, CH, XP), _f32),
            pltpu.VMEM((2, CH), _i32),
            pltpu.VMEM_SHARED((N, D), _f32),
            pltpu.VMEM_SHARED((N, XP), _f32),
            pltpu.SemaphoreType.DMA,
            pltpu.SemaphoreType.DMA,
        ],
        compiler_params=_sc_params,
    )(body)


_sc_scatter_a = _make_sc_scatter((0, 1, 2))
_sc_scatter_b = _make_sc_scatter((3, 4))


def _node_body(h_ref, aggp_ref, cp_ref, xp_ref, wn1h_ref, wn1a_ref, bn1_ref,
               wn2_ref, bn2_ref, ho_ref, xo_ref):
    h = h_ref[...]
    agg = aggp_ref[0] + aggp_ref[1]
    u = jax.nn.silu(jnp.dot(h, wn1h_ref[...], preferred_element_type=_f32)
                    + jnp.dot(agg, wn1a_ref[...], preferred_element_type=_f32)
                    + bn1_ref[...])
    ho_ref[...] = h + jnp.dot(u, wn2_ref[...], preferred_element_type=_f32) \
        + bn2_ref[...]
    s = cp_ref[0] + cp_ref[1]
    cnt = jnp.maximum(s[:, 3:4], 1.0)
    coord = jnp.concatenate(
        [s[:, 0:1], s[:, 8:9], s[:, 1:2],
         jnp.zeros((s.shape[0], XP - 3), _f32)], axis=1)
    xo_ref[...] = xp_ref[...] + coord / cnt


def _node_update(h, aggp, cp, xp, wn1h, wn1a, bn1, wn2, bn2):
    nb = 1000
    grid = N // nb
    full = lambda shp: pl.BlockSpec(shp, lambda i: tuple(0 for _ in shp))
    return pl.pallas_call(
        _node_body,
        grid=(grid,),
        in_specs=[
            pl.BlockSpec((nb, D), lambda i: (i, 0)),
            pl.BlockSpec((NC, nb, D), lambda i: (0, i, 0)),
            pl.BlockSpec((NC, nb, XP), lambda i: (0, i, 0)),
            pl.BlockSpec((nb, XP), lambda i: (i, 0)),
            full((D, D)), full((D, D)), full((1, D)), full((D, D)),
            full((1, D)),
        ],
        out_specs=[
            pl.BlockSpec((nb, D), lambda i: (i, 0)),
            pl.BlockSpec((nb, XP), lambda i: (i, 0)),
        ],
        out_shape=[
            jax.ShapeDtypeStruct((N, D), _f32),
            jax.ShapeDtypeStruct((N, XP), _f32),
        ],
    )(h, aggp, cp, xp, wn1h, wn1a, bn1, wn2, bn2)


def kernel(h, x, edge_index, edge_attr, W_e1, b_e1, W_e2, b_e2, W_n1, b_n1,
           W_n2, b_n2, W_c1, b_c1, W_c2, W_a, b_a):
    xp = jnp.pad(x, ((0, 0), (0, XP - 3)))

    whr = W_e1[:D]
    whc = W_e1[D:2 * D]
    wrad = W_e1[2 * D:2 * D + 1][:, _PERM]
    wea = W_e1[2 * D + 1:][:, _PERM]
    be1 = b_e1[_PERM].reshape(1, D)
    we2 = W_e2[_PERM, :]

    tr, tc = _make_tables(h, xp, whr, whc)
    ms, cvs = [], []
    for s in range(NSLICE):
        gi = _sc_gathers[s](tr, tc, edge_index)
        m, cv = _edge_mlp(s, gi, edge_attr, wea, wrad, be1, we2,
                          b_e2.reshape(1, D), W_a, b_a.reshape(1, 1),
                          W_c1, b_c1.reshape(1, D), W_c2)
        ms.append(m)
        cvs.append(cv)
    zh = jnp.zeros((NC, N, D), _f32)
    zc = jnp.zeros((NC, N, XP), _f32)
    pa_h, pa_c = _sc_scatter_a(ms[0], ms[1], ms[2], cvs[0], cvs[1], cvs[2],
                               edge_index, zh, zc)
    aggp, cp = _sc_scatter_b(ms[3], ms[4], cvs[3], cvs[4],
                             edge_index, pa_h, pa_c)
    ho, xo = _node_update(h, aggp, cp, xp, W_n1[:D], W_n1[D:],
                          b_n1.reshape(1, D), W_n2, b_n2.reshape(1, D))
    return ho, xo[:, :3]


# speedup vs baseline: 1.0379x; 1.0217x over previous
import functools

import jax
import jax.numpy as jnp
import numpy as np
from jax import lax
from jax.experimental import pallas as pl
from jax.experimental.pallas import tpu as pltpu
from jax.experimental.pallas import tpu_sc as plsc

N = 10000
E = 320000
D = 128
XP = 16
TW = 160
GW = TW // 2

NC, NS, L = 2, 16, 16
NW = NC * NS
NSLICE = 5
ES = E // NSLICE
EPW = ES // NW
CH = 80
NCHUNK = EPW // CH
RPS = N // NS

_f32 = jnp.float32
_bf16 = jnp.bfloat16
_i32 = jnp.int32
_mesh = plsc.VectorSubcoreMesh(core_axis_name="c", subcore_axis_name="s")
_sc_params = pltpu.CompilerParams(use_tc_tiling_on_sc=False)
_sc_gather_params = pltpu.CompilerParams(use_tc_tiling_on_sc=False,
                                         needs_layout_passes=False)

_PERM = np.concatenate([np.arange(0, D, 2), np.arange(1, D, 2)])


def _tables_body(h_ref, xp_ref, whr_ref, whc_ref, tr_ref, tc_ref):
    h = h_ref[...]
    xp = xp_ref[...]
    pad = jnp.zeros((h.shape[0], TW - D - XP), _f32)
    tr_ref[...] = jnp.concatenate(
        [jnp.dot(h, whr_ref[...], preferred_element_type=_f32), xp, pad],
        axis=1).astype(_bf16)
    tc_ref[...] = jnp.concatenate(
        [jnp.dot(h, whc_ref[...], preferred_element_type=_f32), xp, pad],
        axis=1).astype(_bf16)


def _make_tables(h, xp, whr, whc):
    nb = 1000
    grid = N // nb
    return pl.pallas_call(
        _tables_body,
        grid=(grid,),
        in_specs=[
            pl.BlockSpec((nb, D), lambda i: (i, 0)),
            pl.BlockSpec((nb, XP), lambda i: (i, 0)),
            pl.BlockSpec((D, D), lambda i: (0, 0)),
            pl.BlockSpec((D, D), lambda i: (0, 0)),
        ],
        out_specs=[
            pl.BlockSpec((nb, TW), lambda i: (i, 0)),
            pl.BlockSpec((nb, TW), lambda i: (i, 0)),
        ],
        out_shape=[
            jax.ShapeDtypeStruct((N, TW), _bf16),
            jax.ShapeDtypeStruct((N, TW), _bf16),
        ],
    )(h, xp, whr, whc)


def _make_sc_gather(s):
    @functools.partial(
        pl.kernel,
        out_type=jax.ShapeDtypeStruct((ES, GW), _i32),
        mesh=_mesh,
        scratch_types=[
            pltpu.VMEM((2, CH), _i32),
            pltpu.VMEM((2, CH), _i32),
            pltpu.VMEM((2, CH, TW), _bf16),
            pltpu.VMEM((2, CH, TW), _bf16),
            pltpu.VMEM((2, CH, GW), _i32),
            pltpu.SemaphoreType.DMA,
            pltpu.SemaphoreType.DMA,
            pltpu.SemaphoreType.DMA,
            pltpu.SemaphoreType.DMA,
            pltpu.SemaphoreType.DMA,
            pltpu.SemaphoreType.DMA,
        ],
        compiler_params=_sc_gather_params,
    )
    def _sc_gather(tr_hbm, tc_hbm, ei_hbm, g_hbm,
                   idxr, idxc, abuf, bbuf, gbuf,
                   sa0, sa1, sb0, sb1, w0, w1):
        wid = lax.axis_index("s") * NC + lax.axis_index("c")
        sa = (sa0, sa1)
        sb = (sb0, sb1)
        ws = (w0, w1)

        def ebase(ci):
            return wid * EPW + ci * CH

        def issue(ci, b):
            base = ebase(ci)
            pltpu.sync_copy(ei_hbm.at[0, pl.ds(s * ES + base, CH)],
                            idxr.at[b])
            pltpu.sync_copy(ei_hbm.at[1, pl.ds(s * ES + base, CH)],
                            idxc.at[b])
            pltpu.async_copy(tr_hbm.at[idxr.at[b]], abuf.at[b], sa[b])
            pltpu.async_copy(tc_hbm.at[idxc.at[b]], bbuf.at[b], sb[b])

        def wait_gather(b):
            pltpu.make_async_copy(tr_hbm.at[idxr.at[b]], abuf.at[b],
                                  sa[b]).wait()
            pltpu.make_async_copy(tc_hbm.at[idxc.at[b]], bbuf.at[b],
                                  sb[b]).wait()

        def wait_write(ci, b):
            base = ebase(ci)
            pltpu.make_async_copy(gbuf.at[b], g_hbm.at[pl.ds(base, CH)],
                                  ws[b]).wait()

        def compute(b):
            @pl.loop(0, CH)
            def _row(i):
                for j in range(TW // 32):
                    sl = pl.ds(32 * j, 32)
                    if j < D // 32:
                        v = abuf[b, i, sl] + bbuf[b, i, sl]
                    else:
                        v = abuf[b, i, sl] - bbuf[b, i, sl]
                    gbuf[b, i, pl.ds(16 * j, 16)] = plsc.bitcast(v, _i32)

        issue(0, 0)
        issue(1, 1)

        @pl.loop(0, NCHUNK - 1, step=2)
        def _chunk(ci):
            for b in (0, 1):
                cur = ci + b
                wait_gather(b)

                @pl.when(cur >= 2)
                def _():
                    wait_write(cur - 2, b)

                compute(b)

                @pl.when(cur + 2 < NCHUNK)
                def _():
                    issue(cur + 2, b)

                pltpu.async_copy(gbuf.at[b],
                                 g_hbm.at[pl.ds(ebase(cur), CH)], ws[b])

        last = NCHUNK - 1
        wait_gather(0)
        wait_write(last - 2, 0)
        compute(0)
        pltpu.sync_copy(gbuf.at[0], g_hbm.at[pl.ds(ebase(last), CH)])
        wait_write(last - 1, 1)

    return _sc_gather


_sc_gathers = [_make_sc_gather(s) for s in range(NSLICE)]


def _edge_body(gi_ref, ea_ref, wea_ref, wrad_ref, be1_ref, we2_ref,
               be2_ref, wa_ref, ba_ref, wc1_ref, bc1_ref, wc2_ref,
               m_ref, cv_ref):
    gi = gi_ref[...]
    lo = jax.lax.bitcast_convert_type(gi << 16, _f32)
    hi = jax.lax.bitcast_convert_type(gi & jnp.int32(-65536), _f32)
    g = jnp.concatenate([lo[:, :D // 2], hi[:, :D // 2]], axis=1)
    d = jnp.concatenate([lo[:, D // 2:D // 2 + XP // 2],
                         hi[:, D // 2:D // 2 + XP // 2]], axis=1)
    ea = ea_ref[...]
    radial = jnp.sum(d * d, axis=1, keepdims=True)
    pre = (g + jnp.dot(ea, wea_ref[...], preferred_element_type=_f32)
           + radial * wrad_ref[...] + be1_ref[...])
    m1 = jax.nn.silu(pre)
    m2 = jax.nn.silu(jnp.dot(m1, we2_ref[...], preferred_element_type=_f32)
                     + be2_ref[...])
    att = jax.nn.sigmoid(jnp.dot(m2, wa_ref[...], preferred_element_type=_f32)
                         + ba_ref[...])
    m = m2 * att
    m_ref[...] = m
    cu = jnp.dot(jax.nn.silu(jnp.dot(m, wc1_ref[...],
                                     preferred_element_type=_f32)
                             + bc1_ref[...]),
                 wc2_ref[...], preferred_element_type=_f32)
    cv = d * cu
    lane = lax.broadcasted_iota(jnp.int32, cv.shape, 1)
    cv_ref[...] = jnp.where(lane == 3, 1.0, cv)


def _edge_mlp(s, gi, ea, wea, wrad, be1, we2, be2, wa, ba, wc1, bc1, wc2):
    eb = 2000
    grid = ES // eb
    off = s * (ES // eb)
    full = lambda shp: pl.BlockSpec(shp, lambda i: tuple(0 for _ in shp))
    return pl.pallas_call(
        _edge_body,
        grid=(grid,),
        in_specs=[
            pl.BlockSpec((eb, GW), lambda i: (i, 0)),
            pl.BlockSpec((eb, 4), lambda i: (i + off, 0)),
            full((4, D)), full((1, D)), full((1, D)), full((D, D)),
            full((1, D)), full((D, 1)), full((1, 1)), full((D, D)),
            full((1, D)), full((D, 1)),
        ],
        out_specs=[
            pl.BlockSpec((eb, D), lambda i: (i, 0)),
            pl.BlockSpec((eb, XP), lambda i: (i, 0)),
        ],
        out_shape=[
            jax.ShapeDtypeStruct((ES, D), _f32),
            jax.ShapeDtypeStruct((ES, XP), _f32),
        ],
    )(gi, ea, wea, wrad, be1, we2, be2, wa, ba, wc1, bc1, wc2)


def _make_sc_scatter(slice_ids):
    nsl = len(slice_ids)

    def body(*refs):
        m_s = refs[0:nsl]
        cv_s = refs[nsl:2 * nsl]
        ei_hbm, inith, initc, aggh_hbm, aggc_hbm = refs[2 * nsl:2 * nsl + 5]
        mbuf, cvbuf, idx, acch, accc, l0, l1 = refs[2 * nsl + 5:]
        cid = lax.axis_index("c")
        sid = lax.axis_index("s")
        wid = sid * NC + cid
        rows = pl.ds(sid * RPS, RPS)
        ls = (l0, l1)

        def issue(p, ci, b):
            base = wid * EPW + ci * CH
            gbase = slice_ids[p] * ES + base
            pltpu.async_copy(ei_hbm.at[0, pl.ds(gbase, CH)], idx.at[b],
                             ls[b])
            pltpu.async_copy(m_s[p].at[pl.ds(base, CH)], mbuf.at[b], ls[b])
            pltpu.async_copy(cv_s[p].at[pl.ds(base, CH)], cvbuf.at[b],
                             ls[b])

        def wait_loads(p, ci, b):
            base = wid * EPW + ci * CH
            gbase = slice_ids[p] * ES + base
            pltpu.make_async_copy(ei_hbm.at[0, pl.ds(gbase, CH)], idx.at[b],
                                  ls[b]).wait()
            pltpu.make_async_copy(m_s[p].at[pl.ds(base, CH)], mbuf.at[b],
                                  ls[b]).wait()
            pltpu.make_async_copy(cv_s[p].at[pl.ds(base, CH)], cvbuf.at[b],
                                  ls[b]).wait()

        def scat(b):
            pltpu.sync_copy(mbuf.at[b], acch.at[idx.at[b]], add=True)
            pltpu.sync_copy(cvbuf.at[b], accc.at[idx.at[b]], add=True)

        issue(0, 0, 0)
        issue(0, 1, 1)
        pltpu.sync_copy(inith.at[cid, rows], acch.at[rows])
        pltpu.sync_copy(initc.at[cid, rows], accc.at[rows])
        plsc.subcore_barrier()

        for p in range(nsl):
            @pl.loop(0, NCHUNK - 1, step=2)
            def _chunk(ci, p=p):
                for b in (0, 1):
                    bb = (b + p) % 2
                    cur = ci + b
                    wait_loads(p, cur, bb)
                    scat(bb)
                    nxt = cur + 2
                    if p + 1 < nsl:
                        @pl.when(nxt < NCHUNK)
                        def _():
                            issue(p, nxt, bb)

                        @pl.when(nxt >= NCHUNK)
                        def _():
                            issue(p + 1, nxt - NCHUNK, bb)
                    else:
                        @pl.when(nxt < NCHUNK)
                        def _():
                            issue(p, nxt, bb)

            last = NCHUNK - 1
            bb = p % 2
            wait_loads(p, last, bb)
            scat(bb)
            if p + 1 < nsl:
                issue(p + 1, 1, bb)

        plsc.subcore_barrier()
        pltpu.sync_copy(acch.at[rows], aggh_hbm.at[cid, rows])
        pltpu.sync_copy(accc.at[rows], aggc_hbm.at[cid, rows])

    return functools.partial(
        pl.kernel,
        out_type=(jax.ShapeDtypeStruct((NC, N, D), _f32),
                  jax.ShapeDtypeStruct((NC, N, XP), _f32)),
        mesh=_mesh,
        scratch_types=[
            pltpu.VMEM((2, CH, D), _f32),
            pltpu.VMEM((2, CH, XP), _f32),
            pltpu.VMEM((2, CH), _i32),
            pltpu.VMEM_SHARED((N, D), _f32),
            pltpu.VMEM_SHARED((N, XP), _f32),
            pltpu.SemaphoreType.DMA,
            pltpu.SemaphoreType.DMA,
        ],
        compiler_params=_sc_params,
    )(body)


_sc_scatter_a = _make_sc_scatter((0, 1, 2, 3))
_sc_scatter_b = _make_sc_scatter((4,))


def _node_body(h_ref, aggp_ref, cp_ref, xp_ref, wn1h_ref, wn1a_ref, bn1_ref,
               wn2_ref, bn2_ref, ho_ref, xo_ref):
    h = h_ref[...]
    agg = aggp_ref[0] + aggp_ref[1]
    u = jax.nn.silu(jnp.dot(h, wn1h_ref[...], preferred_element_type=_f32)
                    + jnp.dot(agg, wn1a_ref[...], preferred_element_type=_f32)
                    + bn1_ref[...])
    ho_ref[...] = h + jnp.dot(u, wn2_ref[...], preferred_element_type=_f32) \
        + bn2_ref[...]
    s = cp_ref[0] + cp_ref[1]
    cnt = jnp.maximum(s[:, 3:4], 1.0)
    coord = jnp.concatenate(
        [s[:, 0:1], s[:, 8:9], s[:, 1:2],
         jnp.zeros((s.shape[0], XP - 3), _f32)], axis=1)
    xo_ref[...] = xp_ref[...] + coord / cnt


def _node_update(h, aggp, cp, xp, wn1h, wn1a, bn1, wn2, bn2):
    nb = 1000
    grid = N // nb
    full = lambda shp: pl.BlockSpec(shp, lambda i: tuple(0 for _ in shp))
    return pl.pallas_call(
        _node_body,
        grid=(grid,),
        in_specs=[
            pl.BlockSpec((nb, D), lambda i: (i, 0)),
            pl.BlockSpec((NC, nb, D), lambda i: (0, i, 0)),
            pl.BlockSpec((NC, nb, XP), lambda i: (0, i, 0)),
            pl.BlockSpec((nb, XP), lambda i: (i, 0)),
            full((D, D)), full((D, D)), full((1, D)), full((D, D)),
            full((1, D)),
        ],
        out_specs=[
            pl.BlockSpec((nb, D), lambda i: (i, 0)),
            pl.BlockSpec((nb, XP), lambda i: (i, 0)),
        ],
        out_shape=[
            jax.ShapeDtypeStruct((N, D), _f32),
            jax.ShapeDtypeStruct((N, XP), _f32),
        ],
    )(h, aggp, cp, xp, wn1h, wn1a, bn1, wn2, bn2)


def kernel(h, x, edge_index, edge_attr, W_e1, b_e1, W_e2, b_e2, W_n1, b_n1,
           W_n2, b_n2, W_c1, b_c1, W_c2, W_a, b_a):
    xp = jnp.pad(x, ((0, 0), (0, XP - 3)))

    whr = W_e1[:D]
    whc = W_e1[D:2 * D]
    wrad = W_e1[2 * D:2 * D + 1][:, _PERM]
    wea = W_e1[2 * D + 1:][:, _PERM]
    be1 = b_e1[_PERM].reshape(1, D)
    we2 = W_e2[_PERM, :]

    tr, tc = _make_tables(h, xp, whr, whc)
    ms, cvs = [], []
    for s in range(NSLICE):
        gi = _sc_gathers[s](tr, tc, edge_index)
        m, cv = _edge_mlp(s, gi, edge_attr, wea, wrad, be1, we2,
                          b_e2.reshape(1, D), W_a, b_a.reshape(1, 1),
                          W_c1, b_c1.reshape(1, D), W_c2)
        ms.append(m)
        cvs.append(cv)
    zh = jnp.zeros((NC, N, D), _f32)
    zc = jnp.zeros((NC, N, XP), _f32)
    pa_h, pa_c = _sc_scatter_a(ms[0], ms[1], ms[2], ms[3],
                               cvs[0], cvs[1], cvs[2], cvs[3],
                               edge_index, zh, zc)
    aggp, cp = _sc_scatter_b(ms[4], cvs[4], edge_index, pa_h, pa_c)
    ho, xo = _node_update(h, aggp, cp, xp, W_n1[:D], W_n1[D:],
                          b_n1.reshape(1, D), W_n2, b_n2.reshape(1, D))
    return ho, xo[:, :3]
